# baseline (speedup 1.0000x reference)
import functools

import jax
import jax.numpy as jnp
from jax import lax
from jax.experimental import pallas as pl
from jax.experimental.pallas import tpu as pltpu
from jax.experimental.pallas import tpu_sc as plsc

M = 2048
D_MODEL = 1024
D_FF = 2048
E = 8
TOP_K = 2

BM = 128
MAXP = TOP_K * M + E * BM
NB = MAXP // BM
KW = D_MODEL // 2

NC, NS = 2, 16
NW = NC * NS
ROWS_W = MAXP // NW
GCH = 80
TOK_W = M // NW
CCH = 16

_NEG = -1e30


def _routing_body(x_ref, wr_ref, gate_ref):
    logits = lax.dot(x_ref[...], wr_ref[...],
                     preferred_element_type=jnp.float32)
    lane = lax.broadcasted_iota(jnp.int32, logits.shape, 1)
    l = jnp.where(lane < E, logits, _NEG)
    m1 = jnp.max(l, axis=1, keepdims=True)
    i1 = jnp.min(jnp.where(l >= m1, lane, 999), axis=1, keepdims=True)
    l2 = jnp.where(lane == i1, _NEG, l)
    m2 = jnp.max(l2, axis=1, keepdims=True)
    i2 = jnp.min(jnp.where(l2 >= m2, lane, 999), axis=1, keepdims=True)
    w0 = 1.0 / (1.0 + jnp.exp(m2 - m1))
    w1 = 1.0 - w0
    gate = jnp.where(lane == i1, w0, 0.0) + jnp.where(lane == i2, w1, 0.0)
    gate_ref[...] = gate[:, :E]


def _gemm_body(be_ref, a_ref, w_ref, ws_ref, c_ref, w16_ref):
    b = pl.program_id(0)
    fresh = jnp.logical_or(b == 0,
                           be_ref[b] != be_ref[jnp.maximum(b - 1, 0)])

    @pl.when(fresh)
    def _():
        w16_ref[...] = w_ref[0].astype(jnp.bfloat16)

    part = lax.dot_general(
        a_ref[...], w16_ref[...],
        dimension_numbers=(((1,), (1,)), ((), ())),
        preferred_element_type=jnp.float32)
    wcol = ws_ref[:, 0:1]
    c_ref[...] = part * wcol


@functools.lru_cache(maxsize=None)
def _build_sc_gather():
    mesh = plsc.VectorSubcoreMesh(core_axis_name="c", subcore_axis_name="s",
                                  num_cores=NC)

    @functools.partial(
        pl.kernel, mesh=mesh,
        out_type=jax.ShapeDtypeStruct((MAXP, KW), jnp.int32),
        scratch_types=[
            pltpu.VMEM((ROWS_W,), jnp.int32),
            pltpu.VMEM((ROWS_W, KW), jnp.int32),
            pltpu.SemaphoreType.DMA,
        ],
    )
    def gather_k(x_hbm, idx_hbm, a_hbm, idx_v, rows_v, sem):
        wid = lax.axis_index("s") * NC + lax.axis_index("c")
        base = wid * ROWS_W
        pltpu.sync_copy(idx_hbm.at[pl.ds(base, ROWS_W)], idx_v)
        cps = []
        for j in range(ROWS_W // GCH):
            cps.append(pltpu.async_copy(
                x_hbm.at[idx_v.at[pl.ds(j * GCH, GCH)]],
                rows_v.at[pl.ds(j * GCH, GCH)], sem))
        for cp in cps:
            cp.wait()
        pltpu.sync_copy(rows_v, a_hbm.at[pl.ds(base, ROWS_W)])

    return gather_k


@functools.lru_cache(maxsize=None)
def _build_sc_combine():
    mesh = plsc.VectorSubcoreMesh(core_axis_name="c", subcore_axis_name="s",
                                  num_cores=NC)

    @functools.partial(
        pl.kernel, mesh=mesh,
        out_type=jax.ShapeDtypeStruct((M, D_FF), jnp.float32),
        scratch_types=[
            pltpu.VMEM((CCH,), jnp.int32),
            pltpu.VMEM((CCH,), jnp.int32),
            pltpu.VMEM((CCH, D_FF), jnp.float32),
            pltpu.VMEM((CCH, D_FF), jnp.float32),
            pltpu.VMEM((CCH, D_FF), jnp.float32),
            pltpu.SemaphoreType.DMA,
        ],
    )
    def combine_k(c_hbm, p0_hbm, p1_hbm, out_hbm,
                  i0_v, i1_v, r0_v, r1_v, o_v, sem):
        wid = lax.axis_index("s") * NC + lax.axis_index("c")
        base = wid * TOK_W
        for c in range(TOK_W // CCH):
            off = base + c * CCH
            pltpu.sync_copy(p0_hbm.at[pl.ds(off, CCH)], i0_v)
            pltpu.sync_copy(p1_hbm.at[pl.ds(off, CCH)], i1_v)
            cp0 = pltpu.async_copy(c_hbm.at[i0_v], r0_v, sem)
            cp1 = pltpu.async_copy(c_hbm.at[i1_v], r1_v, sem)
            cp0.wait()
            cp1.wait()
            for t in range(CCH):
                def _body(j, _, t=t):
                    sl = pl.ds(j * 16, 16)
                    o_v[t, sl] = r0_v[t, sl] + r1_v[t, sl]
                    return 0
                lax.fori_loop(0, D_FF // 16, _body, 0)
            pltpu.sync_copy(o_v, out_hbm.at[pl.ds(off, CCH)])

    return combine_k


def _sc_gather(xi, idx):
    return _build_sc_gather()(xi, idx)


def _sc_combine(c, p0, p1):
    return _build_sc_combine()(c, p0, p1)


def kernel(x, W_router, W_experts):
    wr_pad = jnp.pad(W_router, ((0, 0), (0, 128 - E)))
    gate = pl.pallas_call(
        _routing_body,
        out_shape=jax.ShapeDtypeStruct((M, E), jnp.float32),
    )(x, wr_pad)

    w2, e2 = lax.top_k(gate, TOP_K)
    e_slot = jnp.concatenate([e2[:, 0], e2[:, 1]])
    w_slot = jnp.concatenate([w2[:, 0], w2[:, 1]])
    tok_slot = jnp.concatenate([jnp.arange(M, dtype=jnp.int32)] * 2)
    oh = jax.nn.one_hot(e_slot, E, dtype=jnp.int32)
    csum = jnp.cumsum(oh, axis=0)
    rank = jnp.sum(csum * oh, axis=1) - 1
    counts = csum[-1]
    padded = ((counts + BM - 1) // BM) * BM
    bounds = jnp.cumsum(padded)
    offs = bounds - padded
    dest = (offs[e_slot] + rank).astype(jnp.int32)
    token_of_dest = jnp.zeros((MAXP,), jnp.int32).at[dest].set(tok_slot)
    w_sorted = jnp.zeros((MAXP,), jnp.float32).at[dest].set(w_slot)
    ws2 = jnp.broadcast_to(w_sorted[:, None], (MAXP, 8))
    block_expert = jnp.minimum(
        jnp.searchsorted(bounds, jnp.arange(NB, dtype=jnp.int32) * BM,
                         side="right"),
        E - 1).astype(jnp.int32)
    p0, p1 = dest[:M], dest[M:]

    x16 = x.astype(jnp.bfloat16)
    xi = lax.bitcast_convert_type(
        x16.reshape(M, KW, 2), jnp.int32)

    a_i = _sc_gather(xi, token_of_dest)
    a16 = lax.bitcast_convert_type(a_i, jnp.bfloat16).reshape(MAXP, D_MODEL)

    grid_spec = pltpu.PrefetchScalarGridSpec(
        num_scalar_prefetch=1,
        grid=(NB,),
        in_specs=[
            pl.BlockSpec((BM, D_MODEL), lambda b, be: (b, 0)),
            pl.BlockSpec((1, D_FF, D_MODEL), lambda b, be: (be[b], 0, 0)),
            pl.BlockSpec((BM, 8), lambda b, be: (b, 0)),
        ],
        out_specs=pl.BlockSpec((BM, D_FF), lambda b, be: (b, 0)),
        scratch_shapes=[pltpu.VMEM((D_FF, D_MODEL), jnp.bfloat16)],
    )
    c_sorted = pl.pallas_call(
        _gemm_body,
        grid_spec=grid_spec,
        out_shape=jax.ShapeDtypeStruct((MAXP, D_FF), jnp.float32),
    )(block_expert, a16, W_experts, ws2)

    return _sc_combine(c_sorted, p0, p1)


# speedup vs baseline: 2.0704x; 2.0704x over previous
import functools

import jax
import jax.numpy as jnp
from jax import lax
from jax.experimental import pallas as pl
from jax.experimental.pallas import tpu as pltpu
from jax.experimental.pallas import tpu_sc as plsc

M = 2048
D_MODEL = 1024
D_FF = 2048
E = 8
TOP_K = 2

BM = 128
MAXP = TOP_K * M + E * BM
NB = MAXP // BM

NC, NS = 2, 16
NW = NC * NS
SLOT_W = TOP_K * M // NW
SCH = 64
NSCH = SLOT_W // SCH
TOK_W = M // NW
CCH = 16

_NEG = -1e30


def _routing_body(x_ref, wr_ref, ei_ref, wv_ref):
    logits = lax.dot(x_ref[...], wr_ref[...],
                     preferred_element_type=jnp.float32)
    lane = lax.broadcasted_iota(jnp.int32, logits.shape, 1)
    l = jnp.where(lane < E, logits, _NEG)
    m1 = jnp.max(l, axis=1, keepdims=True)
    i1 = jnp.min(jnp.where(l >= m1, lane, 999), axis=1, keepdims=True)
    l2 = jnp.where(lane == i1, _NEG, l)
    m2 = jnp.max(l2, axis=1, keepdims=True)
    i2 = jnp.min(jnp.where(l2 >= m2, lane, 999), axis=1, keepdims=True)
    w0 = 1.0 / (1.0 + jnp.exp(m2 - m1))
    ei_ref[...] = jnp.concatenate([i1, i2], axis=1)
    wv_ref[...] = jnp.concatenate([w0, 1.0 - w0], axis=1)


def _gemm_body(be_ref, a_ref, w_ref, c_ref, w16_ref):
    b = pl.program_id(0)
    fresh = jnp.logical_or(b == 0,
                           be_ref[b] != be_ref[jnp.maximum(b - 1, 0)])

    @pl.when(fresh)
    def _():
        w16_ref[...] = w_ref[0].astype(jnp.bfloat16)

    c_ref[...] = lax.dot_general(
        a_ref[...].astype(jnp.bfloat16), w16_ref[...],
        dimension_numbers=(((1,), (1,)), ((), ())),
        preferred_element_type=jnp.float32)


@functools.lru_cache(maxsize=None)
def _build_sc_scatter():
    mesh = plsc.VectorSubcoreMesh(core_axis_name="c", subcore_axis_name="s",
                                  num_cores=NC)

    @functools.partial(
        pl.kernel, mesh=mesh,
        out_type=jax.ShapeDtypeStruct((MAXP, D_MODEL), jnp.float32),
        scratch_types=[
            pltpu.VMEM((NSCH, SCH), jnp.int32),
            pltpu.VMEM((SCH, D_MODEL), jnp.float32),
            pltpu.SemaphoreType.DMA,
        ],
    )
    def scatter_k(x_hbm, dest_hbm, a_hbm, idx_v, rows_v, sem):
        wid = lax.axis_index("s") * NC + lax.axis_index("c")
        sbase = wid * SLOT_W
        tbase = sbase % M
        for j in range(NSCH):
            pltpu.sync_copy(dest_hbm.at[pl.ds(sbase + j * SCH, SCH)],
                            idx_v.at[j])
            pltpu.sync_copy(x_hbm.at[pl.ds(tbase + j * SCH, SCH)], rows_v)
            pltpu.async_copy(rows_v, a_hbm.at[idx_v.at[j]], sem).wait()

    return scatter_k


@functools.lru_cache(maxsize=None)
def _build_sc_combine():
    mesh = plsc.VectorSubcoreMesh(core_axis_name="c", subcore_axis_name="s",
                                  num_cores=NC)

    @functools.partial(
        pl.kernel, mesh=mesh,
        out_type=jax.ShapeDtypeStruct((M, D_FF), jnp.float32),
        scratch_types=[
            pltpu.VMEM((CCH,), jnp.int32),
            pltpu.VMEM((CCH,), jnp.int32),
            pltpu.VMEM((CCH, 16), jnp.float32),
            pltpu.VMEM((CCH, 16), jnp.float32),
            pltpu.VMEM((CCH, D_FF), jnp.float32),
            pltpu.VMEM((CCH, D_FF), jnp.float32),
            pltpu.VMEM((CCH, D_FF), jnp.float32),
            pltpu.SemaphoreType.DMA,
        ],
    )
    def combine_k(c_hbm, p0_hbm, p1_hbm, w0_hbm, w1_hbm, out_hbm,
                  i0_v, i1_v, w0_v, w1_v, r0_v, r1_v, o_v, sem):
        wid = lax.axis_index("s") * NC + lax.axis_index("c")
        base = wid * TOK_W
        for c in range(TOK_W // CCH):
            off = base + c * CCH
            pltpu.sync_copy(p0_hbm.at[pl.ds(off, CCH)], i0_v)
            pltpu.sync_copy(p1_hbm.at[pl.ds(off, CCH)], i1_v)
            pltpu.sync_copy(w0_hbm.at[pl.ds(off, CCH)], w0_v)
            pltpu.sync_copy(w1_hbm.at[pl.ds(off, CCH)], w1_v)
            cp0 = pltpu.async_copy(c_hbm.at[i0_v], r0_v, sem)
            cp1 = pltpu.async_copy(c_hbm.at[i1_v], r1_v, sem)
            cp0.wait()
            cp1.wait()
            for t in range(CCH):
                w0t = w0_v[t]
                w1t = w1_v[t]

                def _body(j, _, t=t, w0t=w0t, w1t=w1t):
                    sl = pl.ds(j * 16, 16)
                    o_v[t, sl] = r0_v[t, sl] * w0t + r1_v[t, sl] * w1t
                    return 0
                lax.fori_loop(0, D_FF // 16, _body, 0)
            pltpu.sync_copy(o_v, out_hbm.at[pl.ds(off, CCH)])

    return combine_k


def _sc_scatter(x, dest):
    return _build_sc_scatter()(x, dest)


def _sc_combine(c, p0, p1, w0, w1):
    return _build_sc_combine()(c, p0, p1, w0, w1)


def kernel(x, W_router, W_experts):
    wr_pad = jnp.pad(W_router, ((0, 0), (0, 128 - E)))
    ei, wv = pl.pallas_call(
        _routing_body,
        out_shape=(jax.ShapeDtypeStruct((M, TOP_K), jnp.int32),
                   jax.ShapeDtypeStruct((M, TOP_K), jnp.float32)),
    )(x, wr_pad)

    e_slot = jnp.concatenate([ei[:, 0], ei[:, 1]])
    oh = (e_slot[:, None] == jnp.arange(E, dtype=jnp.int32)[None, :]
          ).astype(jnp.int32)
    csum = jnp.cumsum(oh, axis=0)
    rank = jnp.sum(csum * oh, axis=1) - 1
    counts = csum[-1]
    padded = ((counts + BM - 1) // BM) * BM
    bounds = jnp.cumsum(padded)
    offs = bounds - padded
    dest = (offs[e_slot] + rank).astype(jnp.int32)
    block_expert = jnp.sum(
        (jnp.arange(NB, dtype=jnp.int32)[:, None] * BM >= bounds[None, :]
         ).astype(jnp.int32), axis=1)
    block_expert = jnp.minimum(block_expert, E - 1)
    p0, p1 = dest[:M], dest[M:]

    a_sorted = _sc_scatter(x, dest)

    grid_spec = pltpu.PrefetchScalarGridSpec(
        num_scalar_prefetch=1,
        grid=(NB,),
        in_specs=[
            pl.BlockSpec((BM, D_MODEL), lambda b, be: (b, 0)),
            pl.BlockSpec((1, D_FF, D_MODEL), lambda b, be: (be[b], 0, 0)),
        ],
        out_specs=pl.BlockSpec((BM, D_FF), lambda b, be: (b, 0)),
        scratch_shapes=[pltpu.VMEM((D_FF, D_MODEL), jnp.bfloat16)],
    )
    c_sorted = pl.pallas_call(
        _gemm_body,
        grid_spec=grid_spec,
        out_shape=jax.ShapeDtypeStruct((MAXP, D_FF), jnp.float32),
    )(block_expert, a_sorted, W_experts)

    w0r = jnp.broadcast_to(wv[:, 0:1], (M, 16))
    w1r = jnp.broadcast_to(wv[:, 1:2], (M, 16))
    return _sc_combine(c_sorted, p0, p1, w0r, w1r)


# speedup vs baseline: 2.2797x; 1.1011x over previous
import functools

import jax
import jax.numpy as jnp
from jax import lax
from jax.experimental import pallas as pl
from jax.experimental.pallas import tpu as pltpu
from jax.experimental.pallas import tpu_sc as plsc

M = 2048
D_MODEL = 1024
D_FF = 2048
E = 8
TOP_K = 2

BM = 128
MAXP = TOP_K * M + E * BM
NB = MAXP // BM

NC, NS = 2, 16
NW = NC * NS
SLOT_W = TOP_K * M // NW
SCH = 64
NSCH = SLOT_W // SCH
TOK_W = M // NW
CCH = 8

_NEG = -1e30


def _routing_body(x_ref, wr_ref, ei_ref, wv_ref):
    logits = lax.dot(x_ref[...], wr_ref[...],
                     preferred_element_type=jnp.float32)
    lane = lax.broadcasted_iota(jnp.int32, logits.shape, 1)
    l = jnp.where(lane < E, logits, _NEG)
    m1 = jnp.max(l, axis=1, keepdims=True)
    i1 = jnp.min(jnp.where(l >= m1, lane, 999), axis=1, keepdims=True)
    l2 = jnp.where(lane == i1, _NEG, l)
    m2 = jnp.max(l2, axis=1, keepdims=True)
    i2 = jnp.min(jnp.where(l2 >= m2, lane, 999), axis=1, keepdims=True)
    w0 = 1.0 / (1.0 + jnp.exp(m2 - m1))
    ei_ref[...] = jnp.concatenate([i1, i2], axis=1)
    wv_ref[...] = jnp.concatenate([w0, 1.0 - w0], axis=1)


def _gemm_body(be_ref, a_ref, w_ref, c_ref, w16_ref):
    b = pl.program_id(0)
    fresh = jnp.logical_or(b == 0,
                           be_ref[b] != be_ref[jnp.maximum(b - 1, 0)])

    @pl.when(fresh)
    def _():
        w16_ref[...] = w_ref[0].astype(jnp.bfloat16)

    c_ref[...] = lax.dot_general(
        a_ref[...].astype(jnp.bfloat16), w16_ref[...],
        dimension_numbers=(((1,), (1,)), ((), ())),
        preferred_element_type=jnp.float32)


@functools.lru_cache(maxsize=None)
def _build_sc_scatter():
    mesh = plsc.VectorSubcoreMesh(core_axis_name="c", subcore_axis_name="s",
                                  num_cores=NC)

    @functools.partial(
        pl.kernel, mesh=mesh,
        out_type=jax.ShapeDtypeStruct((MAXP, D_MODEL), jnp.float32),
        scratch_types=[
            pltpu.VMEM((NSCH, SCH), jnp.int32),
            pltpu.VMEM((SCH, D_MODEL), jnp.float32),
            pltpu.SemaphoreType.DMA,
        ],
    )
    def scatter_k(x_hbm, dest_hbm, a_hbm, idx_v, rows_v, sem):
        wid = lax.axis_index("s") * NC + lax.axis_index("c")
        sbase = wid * SLOT_W
        tbase = sbase % M
        for j in range(NSCH):
            pltpu.sync_copy(dest_hbm.at[pl.ds(sbase + j * SCH, SCH)],
                            idx_v.at[j])
            pltpu.sync_copy(x_hbm.at[pl.ds(tbase + j * SCH, SCH)], rows_v)
            pltpu.async_copy(rows_v, a_hbm.at[idx_v.at[j]], sem).wait()

    return scatter_k


@functools.lru_cache(maxsize=None)
def _build_sc_combine():
    mesh = plsc.VectorSubcoreMesh(core_axis_name="c", subcore_axis_name="s",
                                  num_cores=NC)

    NCH = TOK_W // CCH

    @functools.partial(
        pl.kernel, mesh=mesh,
        out_type=jax.ShapeDtypeStruct((M, D_FF), jnp.float32),
        scratch_types=[
            pltpu.VMEM((TOK_W,), jnp.int32),
            pltpu.VMEM((TOK_W,), jnp.int32),
            pltpu.VMEM((TOK_W, 16), jnp.float32),
            pltpu.VMEM((TOK_W, 16), jnp.float32),
            pltpu.VMEM((2, CCH, D_FF), jnp.float32),
            pltpu.VMEM((2, CCH, D_FF), jnp.float32),
            pltpu.VMEM((2, CCH, D_FF), jnp.float32),
            pltpu.SemaphoreType.DMA,
            pltpu.SemaphoreType.DMA,
            pltpu.SemaphoreType.DMA,
            pltpu.SemaphoreType.DMA,
        ],
    )
    def combine_k(c_hbm, p0_hbm, p1_hbm, w0_hbm, w1_hbm, out_hbm,
                  i0_v, i1_v, w0_v, w1_v, r0_v, r1_v, o_v,
                  gs0, gs1, ws0, ws1, ):
        wid = lax.axis_index("s") * NC + lax.axis_index("c")
        base = wid * TOK_W
        pltpu.sync_copy(p0_hbm.at[pl.ds(base, TOK_W)], i0_v)
        pltpu.sync_copy(p1_hbm.at[pl.ds(base, TOK_W)], i1_v)
        pltpu.sync_copy(w0_hbm.at[pl.ds(base, TOK_W)], w0_v)
        pltpu.sync_copy(w1_hbm.at[pl.ds(base, TOK_W)], w1_v)
        gsem = (gs0, gs1)
        wsem = (ws0, ws1)

        def issue_gather(c):
            s = c % 2
            return (pltpu.async_copy(c_hbm.at[i0_v.at[pl.ds(c * CCH, CCH)]],
                                     r0_v.at[s], gsem[s]),
                    pltpu.async_copy(c_hbm.at[i1_v.at[pl.ds(c * CCH, CCH)]],
                                     r1_v.at[s], gsem[s]))

        pending = {0: issue_gather(0)}
        writes = {}
        for c in range(NCH):
            s = c % 2
            if c + 1 < NCH:
                pending[c + 1] = issue_gather(c + 1)
            for cp in pending.pop(c):
                cp.wait()
            if c >= 2:
                writes.pop(c - 2).wait()
            for t in range(CCH):
                w0t = w0_v[c * CCH + t]
                w1t = w1_v[c * CCH + t]

                def _body(j, _, s=s, t=t, w0t=w0t, w1t=w1t):
                    sl = pl.ds(j * 16, 16)
                    o_v[s, t, sl] = r0_v[s, t, sl] * w0t + r1_v[s, t, sl] * w1t
                    return 0
                lax.fori_loop(0, D_FF // 16, _body, 0)
            writes[c] = pltpu.async_copy(
                o_v.at[s], out_hbm.at[pl.ds(base + c * CCH, CCH)], wsem[s])
        for c in sorted(writes):
            writes.pop(c).wait()

    return combine_k


def _sc_scatter(x, dest):
    return _build_sc_scatter()(x, dest)


def _sc_combine(c, p0, p1, w0, w1):
    return _build_sc_combine()(c, p0, p1, w0, w1)


def kernel(x, W_router, W_experts):
    wr_pad = jnp.pad(W_router, ((0, 0), (0, 128 - E)))
    ei, wv = pl.pallas_call(
        _routing_body,
        out_shape=(jax.ShapeDtypeStruct((M, TOP_K), jnp.int32),
                   jax.ShapeDtypeStruct((M, TOP_K), jnp.float32)),
    )(x, wr_pad)

    e_slot = jnp.concatenate([ei[:, 0], ei[:, 1]])
    oh = (e_slot[:, None] == jnp.arange(E, dtype=jnp.int32)[None, :]
          ).astype(jnp.int32)
    csum = jnp.cumsum(oh, axis=0)
    rank = jnp.sum(csum * oh, axis=1) - 1
    counts = csum[-1]
    padded = ((counts + BM - 1) // BM) * BM
    bounds = jnp.cumsum(padded)
    offs = bounds - padded
    dest = (offs[e_slot] + rank).astype(jnp.int32)
    block_expert = jnp.sum(
        (jnp.arange(NB, dtype=jnp.int32)[:, None] * BM >= bounds[None, :]
         ).astype(jnp.int32), axis=1)
    block_expert = jnp.minimum(block_expert, E - 1)
    p0, p1 = dest[:M], dest[M:]

    a_sorted = _sc_scatter(x, dest)

    grid_spec = pltpu.PrefetchScalarGridSpec(
        num_scalar_prefetch=1,
        grid=(NB,),
        in_specs=[
            pl.BlockSpec((BM, D_MODEL), lambda b, be: (b, 0)),
            pl.BlockSpec((1, D_FF, D_MODEL), lambda b, be: (be[b], 0, 0)),
        ],
        out_specs=pl.BlockSpec((BM, D_FF), lambda b, be: (b, 0)),
        scratch_shapes=[pltpu.VMEM((D_FF, D_MODEL), jnp.bfloat16)],
    )
    c_sorted = pl.pallas_call(
        _gemm_body,
        grid_spec=grid_spec,
        out_shape=jax.ShapeDtypeStruct((MAXP, D_FF), jnp.float32),
    )(block_expert, a_sorted, W_experts)

    w0r = jnp.broadcast_to(wv[:, 0:1], (M, 16))
    w1r = jnp.broadcast_to(wv[:, 1:2], (M, 16))
    return _sc_combine(c_sorted, p0, p1, w0r, w1r)


# speedup vs baseline: 2.5710x; 1.1278x over previous
import functools

import jax
import jax.numpy as jnp
from jax import lax
from jax.experimental import pallas as pl
from jax.experimental.pallas import tpu as pltpu
from jax.experimental.pallas import tpu_sc as plsc

M = 2048
D_MODEL = 1024
D_FF = 2048
E = 8
TOP_K = 2

BM = 256
MAXP = TOP_K * M + E * BM
NB = MAXP // BM

NC, NS = 2, 16
NW = NC * NS
SLOT_W = TOP_K * M // NW
SCH = 64
NSCH = SLOT_W // SCH
TOK_W = M // NW
CCH = 8

_NEG = -1e30


def _routing_body(x_ref, wr_ref, ei_ref, wv_ref):
    logits = lax.dot(x_ref[...], wr_ref[...],
                     preferred_element_type=jnp.float32)
    lane = lax.broadcasted_iota(jnp.int32, logits.shape, 1)
    l = jnp.where(lane < E, logits, _NEG)
    m1 = jnp.max(l, axis=1, keepdims=True)
    i1 = jnp.min(jnp.where(l >= m1, lane, 999), axis=1, keepdims=True)
    l2 = jnp.where(lane == i1, _NEG, l)
    m2 = jnp.max(l2, axis=1, keepdims=True)
    i2 = jnp.min(jnp.where(l2 >= m2, lane, 999), axis=1, keepdims=True)
    w0 = 1.0 / (1.0 + jnp.exp(m2 - m1))
    ei_ref[...] = jnp.concatenate([i1, i2], axis=1)
    wv_ref[...] = jnp.concatenate([w0, 1.0 - w0], axis=1)


def _gemm_body(be_ref, a_ref, w_ref, c_ref, w16_ref):
    b = pl.program_id(0)
    fresh = jnp.logical_or(b == 0,
                           be_ref[b] != be_ref[jnp.maximum(b - 1, 0)])

    @pl.when(fresh)
    def _():
        w16_ref[...] = w_ref[0].astype(jnp.bfloat16)

    c_ref[...] = lax.dot_general(
        a_ref[...].astype(jnp.bfloat16), w16_ref[...],
        dimension_numbers=(((1,), (1,)), ((), ())),
        preferred_element_type=jnp.float32)


@functools.lru_cache(maxsize=None)
def _build_sc_scatter():
    mesh = plsc.VectorSubcoreMesh(core_axis_name="c", subcore_axis_name="s",
                                  num_cores=NC)

    @functools.partial(
        pl.kernel, mesh=mesh,
        out_type=jax.ShapeDtypeStruct((MAXP, D_MODEL), jnp.float32),
        scratch_types=[
            pltpu.VMEM((NSCH, SCH), jnp.int32),
            pltpu.VMEM((SCH, D_MODEL), jnp.float32),
            pltpu.SemaphoreType.DMA,
        ],
    )
    def scatter_k(x_hbm, dest_hbm, a_hbm, idx_v, rows_v, sem):
        wid = lax.axis_index("s") * NC + lax.axis_index("c")
        sbase = wid * SLOT_W
        tbase = sbase % M
        for j in range(NSCH):
            pltpu.sync_copy(dest_hbm.at[pl.ds(sbase + j * SCH, SCH)],
                            idx_v.at[j])
            pltpu.sync_copy(x_hbm.at[pl.ds(tbase + j * SCH, SCH)], rows_v)
            pltpu.async_copy(rows_v, a_hbm.at[idx_v.at[j]], sem).wait()

    return scatter_k


@functools.lru_cache(maxsize=None)
def _build_sc_combine():
    mesh = plsc.VectorSubcoreMesh(core_axis_name="c", subcore_axis_name="s",
                                  num_cores=NC)

    NCH = TOK_W // CCH

    @functools.partial(
        pl.kernel, mesh=mesh,
        out_type=jax.ShapeDtypeStruct((M, D_FF), jnp.float32),
        scratch_types=[
            pltpu.VMEM((TOK_W,), jnp.int32),
            pltpu.VMEM((TOK_W,), jnp.int32),
            pltpu.VMEM((TOK_W, 16), jnp.float32),
            pltpu.VMEM((TOK_W, 16), jnp.float32),
            pltpu.VMEM((2, CCH, D_FF), jnp.float32),
            pltpu.VMEM((2, CCH, D_FF), jnp.float32),
            pltpu.VMEM((2, CCH, D_FF), jnp.float32),
            pltpu.SemaphoreType.DMA,
            pltpu.SemaphoreType.DMA,
            pltpu.SemaphoreType.DMA,
            pltpu.SemaphoreType.DMA,
        ],
    )
    def combine_k(c_hbm, p0_hbm, p1_hbm, w0_hbm, w1_hbm, out_hbm,
                  i0_v, i1_v, w0_v, w1_v, r0_v, r1_v, o_v,
                  gs0, gs1, ws0, ws1, ):
        wid = lax.axis_index("s") * NC + lax.axis_index("c")
        base = wid * TOK_W
        pltpu.sync_copy(p0_hbm.at[pl.ds(base, TOK_W)], i0_v)
        pltpu.sync_copy(p1_hbm.at[pl.ds(base, TOK_W)], i1_v)
        pltpu.sync_copy(w0_hbm.at[pl.ds(base, TOK_W)], w0_v)
        pltpu.sync_copy(w1_hbm.at[pl.ds(base, TOK_W)], w1_v)
        gsem = (gs0, gs1)
        wsem = (ws0, ws1)

        def issue_gather(c):
            s = c % 2
            return (pltpu.async_copy(c_hbm.at[i0_v.at[pl.ds(c * CCH, CCH)]],
                                     r0_v.at[s], gsem[s]),
                    pltpu.async_copy(c_hbm.at[i1_v.at[pl.ds(c * CCH, CCH)]],
                                     r1_v.at[s], gsem[s]))

        pending = {0: issue_gather(0)}
        writes = {}
        for c in range(NCH):
            s = c % 2
            if c + 1 < NCH:
                pending[c + 1] = issue_gather(c + 1)
            for cp in pending.pop(c):
                cp.wait()
            if c >= 2:
                writes.pop(c - 2).wait()
            for t in range(CCH):
                w0t = w0_v[c * CCH + t]
                w1t = w1_v[c * CCH + t]

                def _body(j, _, s=s, t=t, w0t=w0t, w1t=w1t):
                    sl = pl.ds(j * 16, 16)
                    o_v[s, t, sl] = r0_v[s, t, sl] * w0t + r1_v[s, t, sl] * w1t
                    return 0
                lax.fori_loop(0, D_FF // 16, _body, 0)
            writes[c] = pltpu.async_copy(
                o_v.at[s], out_hbm.at[pl.ds(base + c * CCH, CCH)], wsem[s])
        for c in sorted(writes):
            writes.pop(c).wait()

    return combine_k


def _sc_scatter(x, dest):
    return _build_sc_scatter()(x, dest)


def _sc_combine(c, p0, p1, w0, w1):
    return _build_sc_combine()(c, p0, p1, w0, w1)


def kernel(x, W_router, W_experts):
    wr_pad = jnp.pad(W_router, ((0, 0), (0, 128 - E)))
    ei, wv = pl.pallas_call(
        _routing_body,
        out_shape=(jax.ShapeDtypeStruct((M, TOP_K), jnp.int32),
                   jax.ShapeDtypeStruct((M, TOP_K), jnp.float32)),
    )(x, wr_pad)

    e_slot = jnp.concatenate([ei[:, 0], ei[:, 1]])
    oh = (e_slot[:, None] == jnp.arange(E, dtype=jnp.int32)[None, :]
          ).astype(jnp.int32)
    csum = jnp.cumsum(oh, axis=0)
    rank = jnp.sum(csum * oh, axis=1) - 1
    counts = csum[-1]
    padded = ((counts + BM - 1) // BM) * BM
    bounds = jnp.cumsum(padded)
    offs = bounds - padded
    dest = (offs[e_slot] + rank).astype(jnp.int32)
    block_expert = jnp.sum(
        (jnp.arange(NB, dtype=jnp.int32)[:, None] * BM >= bounds[None, :]
         ).astype(jnp.int32), axis=1)
    block_expert = jnp.minimum(block_expert, E - 1)
    p0, p1 = dest[:M], dest[M:]

    a_sorted = _sc_scatter(x, dest)

    grid_spec = pltpu.PrefetchScalarGridSpec(
        num_scalar_prefetch=1,
        grid=(NB,),
        in_specs=[
            pl.BlockSpec((BM, D_MODEL), lambda b, be: (b, 0)),
            pl.BlockSpec((1, D_FF, D_MODEL), lambda b, be: (be[b], 0, 0)),
        ],
        out_specs=pl.BlockSpec((BM, D_FF), lambda b, be: (b, 0)),
        scratch_shapes=[pltpu.VMEM((D_FF, D_MODEL), jnp.bfloat16)],
    )
    c_sorted = pl.pallas_call(
        _gemm_body,
        grid_spec=grid_spec,
        out_shape=jax.ShapeDtypeStruct((MAXP, D_FF), jnp.float32),
    )(block_expert, a_sorted, W_experts)

    w0r = jnp.broadcast_to(wv[:, 0:1], (M, 16))
    w1r = jnp.broadcast_to(wv[:, 1:2], (M, 16))
    return _sc_combine(c_sorted, p0, p1, w0r, w1r)


# speedup vs baseline: 2.7052x; 1.0522x over previous
import functools

import jax
import jax.numpy as jnp
from jax import lax
from jax.experimental import pallas as pl
from jax.experimental.pallas import tpu as pltpu
from jax.experimental.pallas import tpu_sc as plsc

M = 2048
D_MODEL = 1024
D_FF = 2048
E = 8
TOP_K = 2

BM = 256
MAXP = TOP_K * M + E * BM
NB = MAXP // BM

NC, NS = 2, 16
NW = NC * NS
SLOT_W = TOP_K * M // NW
SCH = 64
NSCH = SLOT_W // SCH
TOK_W = M // NW
CCH = 8

_NEG = -1e30


def _routing_body(x_ref, wr_ref, wv_ref, dest_ref, bounds_ref):
    logits = lax.dot(x_ref[...], wr_ref[...],
                     preferred_element_type=jnp.float32)
    lane = lax.broadcasted_iota(jnp.int32, logits.shape, 1)
    l = jnp.where(lane < E, logits, _NEG)
    m1 = jnp.max(l, axis=1, keepdims=True)
    i1 = jnp.min(jnp.where(l >= m1, lane, 999), axis=1, keepdims=True)
    l2 = jnp.where(lane == i1, _NEG, l)
    m2 = jnp.max(l2, axis=1, keepdims=True)
    i2 = jnp.min(jnp.where(l2 >= m2, lane, 999), axis=1, keepdims=True)
    w0 = 1.0 / (1.0 + jnp.exp(m2 - m1))
    wv_ref[...] = jnp.concatenate([w0, 1.0 - w0], axis=1)

    oh0 = (lane == i1).astype(jnp.int32)
    oh1 = (lane == i2).astype(jnp.int32)

    def _cumsum_rows(c):
        sh = 1
        while sh < M:
            shifted = jnp.concatenate(
                [jnp.zeros((sh, 128), jnp.int32), c[:M - sh]], axis=0)
            c = c + shifted
            sh *= 2
        return c

    incl0 = _cumsum_rows(oh0)
    incl1 = _cumsum_rows(oh1)
    tot0 = incl0[M - 1:M]
    counts = tot0 + incl1[M - 1:M]
    padded = ((counts + BM - 1) // BM) * BM
    inc = padded
    for d in (1, 2, 4):
        inc = inc + jnp.concatenate(
            [jnp.zeros((1, d), jnp.int32), inc[:, :128 - d]], 1)
    offs = inc - padded
    bounds_ref[...] = inc
    rank0 = jnp.sum(jnp.where(lane == i1, incl0, 0), 1, keepdims=True) - 1
    rank1 = (jnp.sum(jnp.where(lane == i2, incl1 + tot0, 0), 1, keepdims=True)
             - 1)
    off0 = jnp.sum(jnp.where(lane == i1, offs, 0), 1, keepdims=True)
    off1 = jnp.sum(jnp.where(lane == i2, offs, 0), 1, keepdims=True)
    dest_ref[...] = jnp.concatenate([off0 + rank0, off1 + rank1], axis=1)


def _gemm_body(be_ref, a_ref, w_ref, c_ref, w16_ref):
    b = pl.program_id(0)
    fresh = jnp.logical_or(b == 0,
                           be_ref[b] != be_ref[jnp.maximum(b - 1, 0)])

    @pl.when(fresh)
    def _():
        w16_ref[...] = w_ref[0].astype(jnp.bfloat16)

    c_ref[...] = lax.dot_general(
        a_ref[...].astype(jnp.bfloat16), w16_ref[...],
        dimension_numbers=(((1,), (1,)), ((), ())),
        preferred_element_type=jnp.float32)


@functools.lru_cache(maxsize=None)
def _build_sc_scatter():
    mesh = plsc.VectorSubcoreMesh(core_axis_name="c", subcore_axis_name="s",
                                  num_cores=NC)

    @functools.partial(
        pl.kernel, mesh=mesh,
        out_type=jax.ShapeDtypeStruct((MAXP, D_MODEL), jnp.float32),
        scratch_types=[
            pltpu.VMEM((NSCH, SCH), jnp.int32),
            pltpu.VMEM((SCH, D_MODEL), jnp.float32),
            pltpu.SemaphoreType.DMA,
        ],
    )
    def scatter_k(x_hbm, dest_hbm, a_hbm, idx_v, rows_v, sem):
        wid = lax.axis_index("s") * NC + lax.axis_index("c")
        sbase = wid * SLOT_W
        tbase = sbase % M
        for j in range(NSCH):
            pltpu.sync_copy(dest_hbm.at[pl.ds(sbase + j * SCH, SCH)],
                            idx_v.at[j])
            pltpu.sync_copy(x_hbm.at[pl.ds(tbase + j * SCH, SCH)], rows_v)
            pltpu.async_copy(rows_v, a_hbm.at[idx_v.at[j]], sem).wait()

    return scatter_k


@functools.lru_cache(maxsize=None)
def _build_sc_combine():
    mesh = plsc.VectorSubcoreMesh(core_axis_name="c", subcore_axis_name="s",
                                  num_cores=NC)

    NCH = TOK_W // CCH

    @functools.partial(
        pl.kernel, mesh=mesh,
        out_type=jax.ShapeDtypeStruct((M, D_FF), jnp.float32),
        scratch_types=[
            pltpu.VMEM((TOK_W,), jnp.int32),
            pltpu.VMEM((TOK_W,), jnp.int32),
            pltpu.VMEM((TOK_W, 16), jnp.float32),
            pltpu.VMEM((TOK_W, 16), jnp.float32),
            pltpu.VMEM((2, CCH, D_FF), jnp.float32),
            pltpu.VMEM((2, CCH, D_FF), jnp.float32),
            pltpu.VMEM((2, CCH, D_FF), jnp.float32),
            pltpu.SemaphoreType.DMA,
            pltpu.SemaphoreType.DMA,
            pltpu.SemaphoreType.DMA,
            pltpu.SemaphoreType.DMA,
        ],
    )
    def combine_k(c_hbm, p0_hbm, p1_hbm, w0_hbm, w1_hbm, out_hbm,
                  i0_v, i1_v, w0_v, w1_v, r0_v, r1_v, o_v,
                  gs0, gs1, ws0, ws1, ):
        wid = lax.axis_index("s") * NC + lax.axis_index("c")
        base = wid * TOK_W
        pltpu.sync_copy(p0_hbm.at[pl.ds(base, TOK_W)], i0_v)
        pltpu.sync_copy(p1_hbm.at[pl.ds(base, TOK_W)], i1_v)
        pltpu.sync_copy(w0_hbm.at[pl.ds(base, TOK_W)], w0_v)
        pltpu.sync_copy(w1_hbm.at[pl.ds(base, TOK_W)], w1_v)
        gsem = (gs0, gs1)
        wsem = (ws0, ws1)

        def issue_gather(c):
            s = c % 2
            return (pltpu.async_copy(c_hbm.at[i0_v.at[pl.ds(c * CCH, CCH)]],
                                     r0_v.at[s], gsem[s]),
                    pltpu.async_copy(c_hbm.at[i1_v.at[pl.ds(c * CCH, CCH)]],
                                     r1_v.at[s], gsem[s]))

        pending = {0: issue_gather(0)}
        writes = {}
        for c in range(NCH):
            s = c % 2
            if c + 1 < NCH:
                pending[c + 1] = issue_gather(c + 1)
            for cp in pending.pop(c):
                cp.wait()
            if c >= 2:
                writes.pop(c - 2).wait()
            for t in range(CCH):
                w0t = w0_v[c * CCH + t]
                w1t = w1_v[c * CCH + t]

                def _body(j, _, s=s, t=t, w0t=w0t, w1t=w1t):
                    sl = pl.ds(j * 16, 16)
                    o_v[s, t, sl] = r0_v[s, t, sl] * w0t + r1_v[s, t, sl] * w1t
                    return 0
                lax.fori_loop(0, D_FF // 16, _body, 0)
            writes[c] = pltpu.async_copy(
                o_v.at[s], out_hbm.at[pl.ds(base + c * CCH, CCH)], wsem[s])
        for c in sorted(writes):
            writes.pop(c).wait()

    return combine_k


def _sc_scatter(x, dest):
    return _build_sc_scatter()(x, dest)


def _sc_combine(c, p0, p1, w0, w1):
    return _build_sc_combine()(c, p0, p1, w0, w1)


def kernel(x, W_router, W_experts):
    wr_pad = jnp.pad(W_router, ((0, 0), (0, 128 - E)))
    wv, dest2, bounds_row = pl.pallas_call(
        _routing_body,
        out_shape=(jax.ShapeDtypeStruct((M, TOP_K), jnp.float32),
                   jax.ShapeDtypeStruct((M, TOP_K), jnp.int32),
                   jax.ShapeDtypeStruct((1, 128), jnp.int32)),
    )(x, wr_pad)

    bounds = bounds_row[0, :E]
    dest = jnp.concatenate([dest2[:, 0], dest2[:, 1]])
    block_expert = jnp.sum(
        (jnp.arange(NB, dtype=jnp.int32)[:, None] * BM >= bounds[None, :]
         ).astype(jnp.int32), axis=1)
    block_expert = jnp.minimum(block_expert, E - 1)
    p0, p1 = dest2[:, 0], dest2[:, 1]

    a_sorted = _sc_scatter(x, dest)

    grid_spec = pltpu.PrefetchScalarGridSpec(
        num_scalar_prefetch=1,
        grid=(NB,),
        in_specs=[
            pl.BlockSpec((BM, D_MODEL), lambda b, be: (b, 0)),
            pl.BlockSpec((1, D_FF, D_MODEL), lambda b, be: (be[b], 0, 0)),
        ],
        out_specs=pl.BlockSpec((BM, D_FF), lambda b, be: (b, 0)),
        scratch_shapes=[pltpu.VMEM((D_FF, D_MODEL), jnp.bfloat16)],
    )
    c_sorted = pl.pallas_call(
        _gemm_body,
        grid_spec=grid_spec,
        out_shape=jax.ShapeDtypeStruct((MAXP, D_FF), jnp.float32),
    )(block_expert, a_sorted, W_experts)

    w0r = jnp.broadcast_to(wv[:, 0:1], (M, 16))
    w1r = jnp.broadcast_to(wv[:, 1:2], (M, 16))
    return _sc_combine(c_sorted, p0, p1, w0r, w1r)
